# baseline (speedup 1.0000x reference)
import jax
import jax.numpy as jnp
from jax.experimental import pallas as pl
from jax.experimental.pallas import tpu as pltpu

_N = 10000
_C = 128
_CH = 2000
_NCH = _N // _CH


def _body(x_hbm, wxz_ref, wxh_ref, wlin_ref,
          bxz_ref, bhz_ref, bxh_ref, bhh_ref, blin_ref, o_hbm,
          xb, ob, isem, osem):
    wcat = jnp.concatenate([wxz_ref[...], wxh_ref[...]], axis=1)
    bz = bxz_ref[...] + bhz_ref[...]
    bh = bxh_ref[...] + bhh_ref[...]
    wlin = wlin_ref[...]
    blin = blin_ref[...]

    def in_copy(k, slot):
        return pltpu.make_async_copy(
            x_hbm.at[pl.ds(k * _CH, _CH), :], xb.at[slot], isem.at[slot])

    def out_copy(k, slot):
        return pltpu.make_async_copy(
            ob.at[slot], o_hbm.at[pl.ds(k * _CH, _CH), :], osem.at[slot])

    in_copy(0, 0).start()
    for k in range(_NCH):
        slot = k % 2
        if k + 1 < _NCH:
            in_copy(k + 1, (k + 1) % 2).start()
        in_copy(k, slot).wait()
        t = jnp.dot(xb[slot], wcat, preferred_element_type=jnp.float32)
        a = t[:, :_C] + bz
        b = t[:, _C:] + bh
        hpre = (0.5 - 0.5 * jnp.tanh(0.5 * a)) * jnp.tanh(b)
        h = jnp.where(hpre > 0, hpre, jnp.exp(hpre) - 1.0)
        if k >= 2:
            out_copy(k - 2, slot).wait()
        ob[slot] = (
            jnp.dot(h, wlin, preferred_element_type=jnp.float32) + blin
        )
        out_copy(k, slot).start()
    for k in (_NCH - 2, _NCH - 1):
        out_copy(k, k % 2).wait()


def kernel(x, edge_index, edge_weight, W_xz, b_xz, W_hz, b_hz, W_xr, b_xr,
           W_hr, b_hr, W_xh, b_xh, W_hh, b_hh, W_lin, b_lin):
    full = lambda: (0, 0)
    wspec = pl.BlockSpec((_C, _C), lambda: (0, 0))
    bspec = pl.BlockSpec((1, _C), lambda: (0, 0))
    hbmspec = pl.BlockSpec(memory_space=pltpu.MemorySpace.HBM)
    return pl.pallas_call(
        _body,
        grid=(),
        in_specs=[
            hbmspec,
            wspec, wspec, wspec,
            bspec, bspec, bspec, bspec, bspec,
        ],
        out_specs=hbmspec,
        out_shape=jax.ShapeDtypeStruct((_N, _C), jnp.float32),
        scratch_shapes=[
            pltpu.VMEM((2, _CH, _C), jnp.float32),
            pltpu.VMEM((2, _CH, _C), jnp.float32),
            pltpu.SemaphoreType.DMA((2,)),
            pltpu.SemaphoreType.DMA((2,)),
        ],
    )(x, W_xz, W_xh, W_lin,
      b_xz.reshape(1, _C), b_hz.reshape(1, _C),
      b_xh.reshape(1, _C), b_hh.reshape(1, _C), b_lin.reshape(1, _C))


# speedup vs baseline: 1.2571x; 1.2571x over previous
import jax
import jax.numpy as jnp
from jax.experimental import pallas as pl

_N = 10000
_C = 128
_BLK = 5000


def _body(x_ref, wxz_ref, wxh_ref, wlin_ref,
          bxz_ref, bhz_ref, bxh_ref, bhh_ref, blin_ref, o_ref):
    xb = x_ref[...]
    wcat = jnp.concatenate([wxz_ref[...], wxh_ref[...]], axis=1)
    t = jnp.dot(xb, wcat, preferred_element_type=jnp.float32)
    a = t[:, :_C] + (bxz_ref[...] + bhz_ref[...])
    b = t[:, _C:] + (bxh_ref[...] + bhh_ref[...])
    hpre = (0.5 - 0.5 * jnp.tanh(0.5 * a)) * jnp.tanh(b)
    h = jnp.where(hpre > 0, hpre, jnp.exp(hpre) - 1.0)
    o_ref[...] = (
        jnp.dot(h, wlin_ref[...], preferred_element_type=jnp.float32)
        + blin_ref[...]
    )


def kernel(x, edge_index, edge_weight, W_xz, b_xz, W_hz, b_hz, W_xr, b_xr,
           W_hr, b_hr, W_xh, b_xh, W_hh, b_hh, W_lin, b_lin):
    grid = (_N // _BLK,)
    full = lambda i: (0, 0)
    wspec = pl.BlockSpec((_C, _C), full)
    bspec = pl.BlockSpec((1, _C), full)
    return pl.pallas_call(
        _body,
        grid=grid,
        in_specs=[
            pl.BlockSpec((_BLK, _C), lambda i: (i, 0)),
            wspec, wspec, wspec,
            bspec, bspec, bspec, bspec, bspec,
        ],
        out_specs=pl.BlockSpec((_BLK, _C), lambda i: (i, 0)),
        out_shape=jax.ShapeDtypeStruct((_N, _C), jnp.float32),
    )(x, W_xz, W_xh, W_lin,
      b_xz.reshape(1, _C), b_hz.reshape(1, _C),
      b_xh.reshape(1, _C), b_hh.reshape(1, _C), b_lin.reshape(1, _C))
